# baseline (speedup 1.0000x reference)
import jax
import jax.numpy as jnp
from jax.experimental import pallas as pl
from jax.experimental.pallas import tpu as pltpu

B, N, C, FF, E, K = 16, 8, 256, 1024, 8, 2
L = B * N


_CHUNKS = [(0, 3), (3, 6), (6, 8)]


def _moe_kernel(x_ref, rw_ref, b1_ref, b2_ref, w1_hbm, w2_hbm, out_ref,
                w1_buf, w2_buf, sem1, sem2):
    def copies(ci):
        lo, hi = _CHUNKS[ci]
        sl = pl.ds(lo, hi - lo)
        return (pltpu.make_async_copy(w1_hbm.at[sl], w1_buf.at[sl], sem1.at[ci]),
                pltpu.make_async_copy(w2_hbm.at[sl], w2_buf.at[sl], sem2.at[ci]))

    for ci in range(len(_CHUNKS)):
        for c in copies(ci):
            c.start()

    xf = x_ref[:]
    logits = jax.lax.dot_general(
        xf, rw_ref[:], dimension_numbers=(((1,), (1,)), ((), ())),
        preferred_element_type=jnp.float32)
    m = jnp.max(logits, axis=1, keepdims=True)
    ex = jnp.exp(logits - m)
    probs = ex / jnp.sum(ex, axis=1, keepdims=True)
    col = jax.lax.broadcasted_iota(jnp.int32, (L, E), 1)
    p1 = jnp.max(probs, axis=1, keepdims=True)
    i1 = jnp.min(jnp.where(probs == p1, col, E), axis=1, keepdims=True)
    mask1 = col == i1
    pm = jnp.where(mask1, -1.0, probs)
    p2 = jnp.max(pm, axis=1, keepdims=True)
    i2 = jnp.min(jnp.where(pm == p2, col, E), axis=1, keepdims=True)
    mask2 = col == i2
    denom = p1 + p2 + 1e-9
    gates = (jnp.where(mask1, probs, 0.0) + jnp.where(mask2, probs, 0.0)) / denom

    acc = jnp.zeros((L, C), dtype=jnp.float32)
    for ci, (lo, hi) in enumerate(_CHUNKS):
      c1, c2 = copies(ci)
      c1.wait()
      hs = []
      for e in range(lo, hi):
        h = jax.lax.dot_general(
            xf, w1_buf[e], dimension_numbers=(((1,), (1,)), ((), ())),
            preferred_element_type=jnp.float32) + b1_ref[e][None, :]
        hs.append(jnp.maximum(h, 0.0))
      c2.wait()
      for e in range(lo, hi):
        o = jax.lax.dot_general(
            hs[e - lo], w2_buf[e], dimension_numbers=(((1,), (1,)), ((), ())),
            preferred_element_type=jnp.float32) + b2_ref[e][None, :]
        acc = acc + gates[:, e:e + 1] * o
    out_ref[:] = acc


def kernel(x, router_w, w1_all, b1_all, w2_all, b2_all):
    xf = x.reshape(L, C)
    out = pl.pallas_call(
        _moe_kernel,
        in_specs=[
            pl.BlockSpec(memory_space=pltpu.MemorySpace.VMEM),
            pl.BlockSpec(memory_space=pltpu.MemorySpace.VMEM),
            pl.BlockSpec(memory_space=pltpu.MemorySpace.VMEM),
            pl.BlockSpec(memory_space=pltpu.MemorySpace.VMEM),
            pl.BlockSpec(memory_space=pl.ANY),
            pl.BlockSpec(memory_space=pl.ANY),
        ],
        out_specs=pl.BlockSpec(memory_space=pltpu.MemorySpace.VMEM),
        out_shape=jax.ShapeDtypeStruct((L, C), jnp.float32),
        scratch_shapes=[
            pltpu.VMEM((E, FF, C), jnp.float32),
            pltpu.VMEM((E, C, FF), jnp.float32),
            pltpu.SemaphoreType.DMA((len(_CHUNKS),)),
            pltpu.SemaphoreType.DMA((len(_CHUNKS),)),
        ],
    )(xf, router_w, b1_all, b2_all, w1_all, w2_all)
    return out.reshape(B, N, C)


# speedup vs baseline: 1.0210x; 1.0210x over previous
import jax
import jax.numpy as jnp
from jax.experimental import pallas as pl
from jax.experimental.pallas import tpu as pltpu

B, N, C, FF, E, K = 16, 8, 256, 1024, 8, 2
L = B * N


_CHUNKS = [(0, 2), (2, 4), (4, 6), (6, 8)]


def _moe_kernel(x_ref, rw_ref, b1_ref, b2_ref, w1_hbm, w2_hbm, out_ref,
                w1_buf, w2_buf, sem1, sem2):
    def copies(ci):
        lo, hi = _CHUNKS[ci]
        sl = pl.ds(lo, hi - lo)
        return (pltpu.make_async_copy(w1_hbm.at[sl], w1_buf.at[sl], sem1.at[ci]),
                pltpu.make_async_copy(w2_hbm.at[sl], w2_buf.at[sl], sem2.at[ci]))

    for ci in range(len(_CHUNKS)):
        for c in copies(ci):
            c.start()

    xf = x_ref[:]
    logits = jax.lax.dot_general(
        xf, rw_ref[:], dimension_numbers=(((1,), (1,)), ((), ())),
        preferred_element_type=jnp.float32)
    m = jnp.max(logits, axis=1, keepdims=True)
    ex = jnp.exp(logits - m)
    probs = ex / jnp.sum(ex, axis=1, keepdims=True)
    col = jax.lax.broadcasted_iota(jnp.int32, (L, E), 1)
    p1 = jnp.max(probs, axis=1, keepdims=True)
    i1 = jnp.min(jnp.where(probs == p1, col, E), axis=1, keepdims=True)
    mask1 = col == i1
    pm = jnp.where(mask1, -1.0, probs)
    p2 = jnp.max(pm, axis=1, keepdims=True)
    i2 = jnp.min(jnp.where(pm == p2, col, E), axis=1, keepdims=True)
    mask2 = col == i2
    denom = p1 + p2 + 1e-9
    gates = (jnp.where(mask1, probs, 0.0) + jnp.where(mask2, probs, 0.0)) / denom

    acc = jnp.zeros((L, C), dtype=jnp.float32)
    for ci, (lo, hi) in enumerate(_CHUNKS):
      c1, c2 = copies(ci)
      c1.wait()
      hs = []
      for e in range(lo, hi):
        h = jax.lax.dot_general(
            xf, w1_buf[e], dimension_numbers=(((1,), (1,)), ((), ())),
            preferred_element_type=jnp.float32) + b1_ref[e][None, :]
        hs.append(jnp.maximum(h, 0.0))
      c2.wait()
      for e in range(lo, hi):
        o = jax.lax.dot_general(
            hs[e - lo], w2_buf[e], dimension_numbers=(((1,), (1,)), ((), ())),
            preferred_element_type=jnp.float32) + b2_ref[e][None, :]
        acc = acc + gates[:, e:e + 1] * o
    out_ref[:] = acc


def kernel(x, router_w, w1_all, b1_all, w2_all, b2_all):
    xf = x.reshape(L, C)
    out = pl.pallas_call(
        _moe_kernel,
        in_specs=[
            pl.BlockSpec(memory_space=pltpu.MemorySpace.VMEM),
            pl.BlockSpec(memory_space=pltpu.MemorySpace.VMEM),
            pl.BlockSpec(memory_space=pltpu.MemorySpace.VMEM),
            pl.BlockSpec(memory_space=pltpu.MemorySpace.VMEM),
            pl.BlockSpec(memory_space=pl.ANY),
            pl.BlockSpec(memory_space=pl.ANY),
        ],
        out_specs=pl.BlockSpec(memory_space=pltpu.MemorySpace.VMEM),
        out_shape=jax.ShapeDtypeStruct((L, C), jnp.float32),
        scratch_shapes=[
            pltpu.VMEM((E, FF, C), jnp.float32),
            pltpu.VMEM((E, C, FF), jnp.float32),
            pltpu.SemaphoreType.DMA((len(_CHUNKS),)),
            pltpu.SemaphoreType.DMA((len(_CHUNKS),)),
        ],
    )(xf, router_w, b1_all, b2_all, w1_all, w2_all)
    return out.reshape(B, N, C)
